# Initial kernel scaffold; baseline (speedup 1.0000x reference)
import jax
import jax.numpy as jnp
from jax.experimental import pallas as pl


def kernel(x, emb_proto, emb_flags, emb_dir, W_len, b_len, W_iat, b_iat, W_fusion, b_fusion, gamma, beta):
    raise NotImplementedError("write your pallas kernel here")


# speedup vs baseline: 8.5986x; 8.5986x over previous
import functools

import jax
import jax.numpy as jnp
from jax.experimental import pallas as pl

_D = 256
_NP = 256
_NFD = 128
_R = 1024


def _tables_kernel(ep_ref, ef_ref, ed_ref, wft_ref, wlt_ref, bl_ref, wit_ref,
                   bi_ref, bf_ref, tp_ref, tfd_ref, vl_ref, vi_ref):
    wft = wft_ref[...]
    wp = wft[0:32, :]
    wl = wft[32:64, :]
    wf = wft[64:96, :]
    wi = wft[96:128, :]
    wd = wft[128:136, :]
    f32 = jnp.float32
    tp = jnp.dot(ep_ref[...], wp, preferred_element_type=f32)
    tf = jnp.dot(ef_ref[...], wf, preferred_element_type=f32)
    td = jnp.dot(ed_ref[...], wd, preferred_element_type=f32)
    cc = (jnp.dot(bl_ref[...], wl, preferred_element_type=f32)
          + jnp.dot(bi_ref[...], wi, preferred_element_type=f32)
          + bf_ref[...])
    tfc = tf + cc
    tfd = jnp.concatenate([tfc + td[0:1, :], tfc + td[1:2, :]], axis=0)
    tp_ref[...] = tp.astype(jnp.bfloat16)
    tfd_ref[...] = tfd.astype(jnp.bfloat16)
    vl_ref[...] = jnp.dot(wlt_ref[...], wl, preferred_element_type=f32)
    vi_ref[...] = jnp.dot(wit_ref[...], wi, preferred_element_type=f32)


def _main_kernel(x_ref, tp_ref, tfd_ref, vl_ref, vi_ref, gm_ref, bt_ref, o_ref):
    xb = x_ref[...]
    x1 = xb[:, 1:2]
    x3 = xb[:, 3:4]
    ip = jnp.clip(xb[:, 0:1].astype(jnp.int32), 0, 255)
    ifd = (jnp.clip(xb[:, 2:3].astype(jnp.int32), 0, 63)
           + 64 * jnp.clip(xb[:, 4:5].astype(jnp.int32), 0, 1))
    iota_p = jax.lax.broadcasted_iota(jnp.int32, (1, _NP), 1)
    iota_f = jax.lax.broadcasted_iota(jnp.int32, (1, _NFD), 1)
    ohp = (ip == iota_p).astype(jnp.bfloat16)
    ohf = (ifd == iota_f).astype(jnp.bfloat16)
    acc = jnp.dot(ohp, tp_ref[...], preferred_element_type=jnp.float32)
    acc = acc + jnp.dot(ohf, tfd_ref[...], preferred_element_type=jnp.float32)
    h = acc + x1 * vl_ref[...] + x3 * vi_ref[...]
    mu = jnp.mean(h, axis=1, keepdims=True)
    d = h - mu
    var = jnp.mean(d * d, axis=1, keepdims=True)
    o_ref[...] = d * jax.lax.rsqrt(var + 1e-5) * gm_ref[...] + bt_ref[...]


@functools.partial(jax.jit, static_argnames=())
def kernel(x, emb_proto, emb_flags, emb_dir, W_len, b_len, W_iat, b_iat,
           W_fusion, b_fusion, gamma, beta):
    B, L, _ = x.shape
    n = B * L
    x2d = x.reshape(n, 5)
    f32 = jnp.float32
    tp, tfd, vl, vi = pl.pallas_call(
        _tables_kernel,
        out_shape=(
            jax.ShapeDtypeStruct((_NP, _D), jnp.bfloat16),
            jax.ShapeDtypeStruct((_NFD, _D), jnp.bfloat16),
            jax.ShapeDtypeStruct((1, _D), f32),
            jax.ShapeDtypeStruct((1, _D), f32),
        ),
    )(emb_proto, emb_flags, emb_dir, W_fusion.T,
      W_len.T, b_len.reshape(1, -1), W_iat.T, b_iat.reshape(1, -1),
      b_fusion.reshape(1, -1))

    grid = n // _R
    out = pl.pallas_call(
        _main_kernel,
        grid=(grid,),
        in_specs=[
            pl.BlockSpec((_R, 5), lambda i: (i, 0)),
            pl.BlockSpec((_NP, _D), lambda i: (0, 0)),
            pl.BlockSpec((_NFD, _D), lambda i: (0, 0)),
            pl.BlockSpec((1, _D), lambda i: (0, 0)),
            pl.BlockSpec((1, _D), lambda i: (0, 0)),
            pl.BlockSpec((1, _D), lambda i: (0, 0)),
            pl.BlockSpec((1, _D), lambda i: (0, 0)),
        ],
        out_specs=pl.BlockSpec((_R, _D), lambda i: (i, 0)),
        out_shape=jax.ShapeDtypeStruct((n, _D), f32),
    )(x2d, tp, tfd, vl, vi, gamma.reshape(1, -1), beta.reshape(1, -1))
    return out.reshape(B, L, _D)
